# baseline (speedup 1.0000x reference)
import functools

import jax
import jax.numpy as jnp
from jax import lax
from jax.experimental import pallas as pl
from jax.experimental.pallas import tpu as pltpu
from jax.experimental.pallas import tpu_sc as plsc

NUM_CORES = 2
NUM_SUBCORES = 16
NW = NUM_CORES * NUM_SUBCORES
NROWS = 1000000
BATCH = 16384
EMB = 64
ROWPAD = 128
BPW = BATCH // NW
HALF = BPW // 2
IDC = 128
LANES = 16
NGROUP = HALF // LANES

PAD_BN = 10000


def _pad_body(u_ref, i_ref, uo_ref, io_ref):
    xu = u_ref[...]
    xi = i_ref[...]
    uo_ref[...] = jnp.concatenate([xu, xu], axis=1)
    io_ref[...] = jnp.concatenate([xi, xi], axis=1)


@jax.jit
def _pad_tables(user_emb, item_emb):
    return pl.pallas_call(
        _pad_body,
        grid=(NROWS // PAD_BN,),
        in_specs=[
            pl.BlockSpec((PAD_BN, EMB), lambda i: (i, 0)),
            pl.BlockSpec((PAD_BN, EMB), lambda i: (i, 0)),
        ],
        out_specs=[
            pl.BlockSpec((PAD_BN, ROWPAD), lambda i: (i, 0)),
            pl.BlockSpec((PAD_BN, ROWPAD), lambda i: (i, 0)),
        ],
        out_shape=[
            jax.ShapeDtypeStruct((NROWS, ROWPAD), jnp.float32),
            jax.ShapeDtypeStruct((NROWS, ROWPAD), jnp.float32),
        ],
    )(user_emb, item_emb)


def _bpr_body(user_pad, item_pad, user_ids, pos_item_ids, neg_item_ids,
              pos_out, neg_out,
              uid_v, pid_v, nid_v, u_v, p_v, n_v, po_v, no_v, sem):
    wid = lax.axis_index("s") * NUM_CORES + lax.axis_index("c")
    base = wid * BPW

    pltpu.sync_copy(user_ids.at[pl.ds(base, BPW)], uid_v)
    pltpu.sync_copy(pos_item_ids.at[pl.ds(base, BPW)], pid_v)
    pltpu.sync_copy(neg_item_ids.at[pl.ds(base, BPW)], nid_v)

    lanes = lax.iota(jnp.int32, LANES)

    for h in range(2):
        hoff = h * HALF
        copies = []
        for k in range(HALF // IDC):
            src = pl.ds(hoff + k * IDC, IDC)
            dst = pl.ds(k * IDC, IDC)
            copies.append(pltpu.async_copy(
                user_pad.at[uid_v.at[src]], u_v.at[dst], sem))
            copies.append(pltpu.async_copy(
                item_pad.at[pid_v.at[src]], p_v.at[dst], sem))
            copies.append(pltpu.async_copy(
                item_pad.at[nid_v.at[src]], n_v.at[dst], sem))
        for cp in copies:
            cp.wait()

        def group_step(g, carry):
            rows = g * LANES + lanes
            acc_p = jnp.zeros((LANES,), jnp.float32)
            acc_n = jnp.zeros((LANES,), jnp.float32)
            for d in range(EMB):
                cols = jnp.full((LANES,), d, jnp.int32)
                uu = plsc.load_gather(u_v, [rows, cols])
                pp = plsc.load_gather(p_v, [rows, cols])
                nn = plsc.load_gather(n_v, [rows, cols])
                acc_p = acc_p + uu * pp
                acc_n = acc_n + uu * nn
            po_v[pl.ds(hoff + g * LANES, LANES)] = acc_p
            no_v[pl.ds(hoff + g * LANES, LANES)] = acc_n
            return carry

        lax.fori_loop(0, NGROUP, group_step, 0)

    pltpu.sync_copy(po_v, pos_out.at[pl.ds(base, BPW)])
    pltpu.sync_copy(no_v, neg_out.at[pl.ds(base, BPW)])


@jax.jit
def _bpr(user_emb, item_emb, user_ids, pos_item_ids, neg_item_ids):
    user_pad, item_pad = _pad_tables(user_emb, item_emb)
    mesh = plsc.VectorSubcoreMesh(core_axis_name="c", subcore_axis_name="s")
    run = functools.partial(
        pl.kernel,
        out_type=(
            jax.ShapeDtypeStruct((BATCH,), jnp.float32),
            jax.ShapeDtypeStruct((BATCH,), jnp.float32),
        ),
        mesh=mesh,
        scratch_types=[
            pltpu.VMEM((BPW,), jnp.int32),
            pltpu.VMEM((BPW,), jnp.int32),
            pltpu.VMEM((BPW,), jnp.int32),
            pltpu.VMEM((HALF, ROWPAD), jnp.float32),
            pltpu.VMEM((HALF, ROWPAD), jnp.float32),
            pltpu.VMEM((HALF, ROWPAD), jnp.float32),
            pltpu.VMEM((BPW,), jnp.float32),
            pltpu.VMEM((BPW,), jnp.float32),
            pltpu.SemaphoreType.DMA,
        ],
        compiler_params=pltpu.CompilerParams(needs_layout_passes=False),
    )(_bpr_body)
    return run(user_pad, item_pad, user_ids, pos_item_ids, neg_item_ids)


def kernel(user_emb, item_emb, user_ids, pos_item_ids, neg_item_ids):
    return _bpr(user_emb, item_emb,
                user_ids.astype(jnp.int32),
                pos_item_ids.astype(jnp.int32),
                neg_item_ids.astype(jnp.int32))


# speedup vs baseline: 1.1987x; 1.1987x over previous
import functools

import jax
import jax.numpy as jnp
from jax import lax
from jax.experimental import pallas as pl
from jax.experimental.pallas import tpu as pltpu
from jax.experimental.pallas import tpu_sc as plsc

NUM_CORES = 2
NUM_SUBCORES = 16
NW = NUM_CORES * NUM_SUBCORES
NROWS = 1000000
NPAIR = NROWS // 2
BATCH = 16384
EMB = 64
ROWPAD = 128
BPW = BATCH // NW
HALF = BPW // 2
IDC = 128
LANES = 16
NGROUP = HALF // LANES


def _bpr_body(user2, item2, user_ids, pos_item_ids, neg_item_ids,
              pos_out, neg_out,
              uid_v, pid_v, nid_v, utid_v, ptid_v, ntid_v,
              u_v, p_v, n_v, po_v, no_v, sem):
    wid = lax.axis_index("s") * NUM_CORES + lax.axis_index("c")
    base = wid * BPW

    pltpu.sync_copy(user_ids.at[pl.ds(base, BPW)], uid_v)
    pltpu.sync_copy(pos_item_ids.at[pl.ds(base, BPW)], pid_v)
    pltpu.sync_copy(neg_item_ids.at[pl.ds(base, BPW)], nid_v)

    def tid_step(k, carry):
        s = pl.ds(k * LANES, LANES)
        utid_v[s] = uid_v[s] >> 1
        ptid_v[s] = pid_v[s] >> 1
        ntid_v[s] = nid_v[s] >> 1
        return carry

    lax.fori_loop(0, BPW // LANES, tid_step, 0)

    lanes = lax.iota(jnp.int32, LANES)

    for h in range(2):
        hoff = h * HALF
        copies = []
        for k in range(HALF // IDC):
            src = pl.ds(hoff + k * IDC, IDC)
            dst = pl.ds(k * IDC, IDC)
            copies.append(pltpu.async_copy(
                user2.at[utid_v.at[src]], u_v.at[dst], sem))
            copies.append(pltpu.async_copy(
                item2.at[ptid_v.at[src]], p_v.at[dst], sem))
            copies.append(pltpu.async_copy(
                item2.at[ntid_v.at[src]], n_v.at[dst], sem))
        for cp in copies:
            cp.wait()

        def group_step(g, carry):
            pos = pl.ds(hoff + g * LANES, LANES)
            rows = g * LANES + lanes
            ucol = (uid_v[pos] & 1) * EMB
            pcol = (pid_v[pos] & 1) * EMB
            ncol = (nid_v[pos] & 1) * EMB
            acc_p = jnp.zeros((LANES,), jnp.float32)
            acc_n = jnp.zeros((LANES,), jnp.float32)
            for d in range(EMB):
                uu = plsc.load_gather(u_v, [rows, ucol + d])
                pp = plsc.load_gather(p_v, [rows, pcol + d])
                nn = plsc.load_gather(n_v, [rows, ncol + d])
                acc_p = acc_p + uu * pp
                acc_n = acc_n + uu * nn
            po_v[pos] = acc_p
            no_v[pos] = acc_n
            return carry

        lax.fori_loop(0, NGROUP, group_step, 0)

    pltpu.sync_copy(po_v, pos_out.at[pl.ds(base, BPW)])
    pltpu.sync_copy(no_v, neg_out.at[pl.ds(base, BPW)])


@jax.jit
def _bpr(user_emb, item_emb, user_ids, pos_item_ids, neg_item_ids):
    user2 = jnp.reshape(user_emb, (NPAIR, ROWPAD))
    item2 = jnp.reshape(item_emb, (NPAIR, ROWPAD))
    mesh = plsc.VectorSubcoreMesh(core_axis_name="c", subcore_axis_name="s")
    run = functools.partial(
        pl.kernel,
        out_type=(
            jax.ShapeDtypeStruct((BATCH,), jnp.float32),
            jax.ShapeDtypeStruct((BATCH,), jnp.float32),
        ),
        mesh=mesh,
        scratch_types=[
            pltpu.VMEM((BPW,), jnp.int32),
            pltpu.VMEM((BPW,), jnp.int32),
            pltpu.VMEM((BPW,), jnp.int32),
            pltpu.VMEM((BPW,), jnp.int32),
            pltpu.VMEM((BPW,), jnp.int32),
            pltpu.VMEM((BPW,), jnp.int32),
            pltpu.VMEM((HALF, ROWPAD), jnp.float32),
            pltpu.VMEM((HALF, ROWPAD), jnp.float32),
            pltpu.VMEM((HALF, ROWPAD), jnp.float32),
            pltpu.VMEM((BPW,), jnp.float32),
            pltpu.VMEM((BPW,), jnp.float32),
            pltpu.SemaphoreType.DMA,
        ],
        compiler_params=pltpu.CompilerParams(needs_layout_passes=False),
    )(_bpr_body)
    return run(user2, item2, user_ids, pos_item_ids, neg_item_ids)


def kernel(user_emb, item_emb, user_ids, pos_item_ids, neg_item_ids):
    return _bpr(user_emb, item_emb,
                user_ids.astype(jnp.int32),
                pos_item_ids.astype(jnp.int32),
                neg_item_ids.astype(jnp.int32))
